# baseline (device time: 15257 ns/iter reference)
import jax
import jax.numpy as jnp
from jax import lax
from jax.experimental import pallas as pl
from jax.experimental.pallas import tpu as pltpu

N_DEV = 16
GRID = 8


def kernel(x, dy, gamma):
    m, d = x.shape
    bm = m // GRID

    def body(x_ref, dy_ref, gamma_ref, out_ref, acc_ref, send_buf, recv_buf,
             send_sems, recv_sems):
        my_pos = lax.axis_index("i")
        g = pl.program_id(0)

        barrier_sem = pltpu.get_barrier_semaphore()

        @pl.when(g == 0)
        def _():
            for dd in range(1, N_DEV):
                peer = lax.rem(my_pos + dd, N_DEV)
                pl.semaphore_signal(
                    barrier_sem, inc=1,
                    device_id=(peer,), device_id_type=pl.DeviceIdType.MESH,
                )

        xv = x_ref[:, :]
        dyv = dy_ref[:, :]
        mu = jnp.mean(xv, axis=1, keepdims=True)
        var = jnp.mean(xv * xv, axis=1, keepdims=True) - mu * mu
        rstd = lax.rsqrt(var + 1e-5)
        xhat = (xv - mu) * rstd
        dgamma = jnp.sum(dyv * xhat, axis=0, keepdims=True)
        dbeta = jnp.sum(dyv, axis=0, keepdims=True)
        partial = jnp.concatenate([dgamma, dbeta], axis=0)

        @pl.when(g == 0)
        def _():
            acc_ref[:, :] = partial

        @pl.when(g != 0)
        def _():
            acc_ref[:, :] = acc_ref[:, :] + partial

        @pl.when(g == GRID - 1)
        def _():
            mine = acc_ref[:, :]
            send_buf[:, :] = mine
            pl.semaphore_wait(barrier_sem, N_DEV - 1)

            rdmas = []
            for dd in range(1, N_DEV):
                peer = lax.rem(my_pos + dd, N_DEV)
                rdma = pltpu.make_async_remote_copy(
                    src_ref=send_buf,
                    dst_ref=recv_buf.at[dd - 1],
                    send_sem=send_sems.at[dd - 1],
                    recv_sem=recv_sems.at[dd - 1],
                    device_id=(peer,),
                    device_id_type=pl.DeviceIdType.MESH,
                )
                rdma.start()
                rdmas.append(rdma)

            for rdma in rdmas:
                rdma.wait_recv()
            out_ref[:, :] = mine + jnp.sum(recv_buf[:, :, :], axis=0)

            for rdma in rdmas:
                rdma.wait_send()

    return pl.pallas_call(
        body,
        grid=(GRID,),
        out_shape=jax.ShapeDtypeStruct((2, d), jnp.float32),
        in_specs=[
            pl.BlockSpec((bm, d), lambda g: (g, 0)),
            pl.BlockSpec((bm, d), lambda g: (g, 0)),
            pl.BlockSpec((d,), lambda g: (0,)),
        ],
        out_specs=pl.BlockSpec((2, d), lambda g: (0, 0)),
        scratch_shapes=[
            pltpu.VMEM((2, d), jnp.float32),
            pltpu.VMEM((2, d), jnp.float32),
            pltpu.VMEM((N_DEV - 1, 2, d), jnp.float32),
            pltpu.SemaphoreType.DMA((N_DEV - 1,)),
            pltpu.SemaphoreType.DMA((N_DEV - 1,)),
        ],
        compiler_params=pltpu.CompilerParams(collective_id=0),
    )(x, dy, gamma)
